# baseline (speedup 1.0000x reference)
import dataclasses
import functools

import jax
import jax.numpy as jnp
from jax import lax
from jax.experimental import pallas as pl
from jax.experimental.pallas import tpu as pltpu
from jax.experimental.pallas import tpu_sc as plsc

DIM = 128
N_LAYERS = 3
N_ITER = 4


BM = 1000


def _mlp2_body(x_ref, Wp_ref, bp_ref, Wn_ref, bn_ref, op_ref, on_ref):
    x = x_ref[...]

    def run(W_ref, b_ref, o_ref):
        h = x
        for i in range(N_LAYERS):
            h = lax.dot_general(h, W_ref[i], (((1,), (1,)), ((), ())),
                                preferred_element_type=jnp.float32)
            h = h + b_ref[i][None, :]
            if i < N_LAYERS - 1:
                h = jnp.maximum(h, 0.0)
        o_ref[...] = h

    run(Wp_ref, bp_ref, op_ref)
    run(Wn_ref, bn_ref, on_ref)


def _mlp2(x, Wp, bp, Wn, bn):
    n = x.shape[0]
    bs_x = pl.BlockSpec((BM, DIM), lambda i: (i, 0))
    bs_w = pl.BlockSpec((N_LAYERS, DIM, DIM), lambda i: (0, 0, 0))
    bs_b = pl.BlockSpec((N_LAYERS, DIM), lambda i: (0, 0))
    return pl.pallas_call(
        _mlp2_body,
        grid=(n // BM,),
        in_specs=[bs_x, bs_w, bs_b, bs_w, bs_b],
        out_specs=[bs_x, bs_x],
        out_shape=[jax.ShapeDtypeStruct((n, DIM), jnp.float32)] * 2,
    )(x, Wp, bp, Wn, bn)


def _gru_body(p_ref, n_ref, h_ref, Wih_ref, Whh_ref, bih_ref, bhh_ref, o_ref):
    p = p_ref[...]
    q = n_ref[...]
    h = h_ref[...]
    Wih = Wih_ref[...]
    Whh = Whh_ref[...]
    dn = (((1,), (1,)), ((), ()))
    gi = lax.dot_general(p, Wih[:, :DIM], dn, preferred_element_type=jnp.float32)
    gi = gi + lax.dot_general(q, Wih[:, DIM:], dn, preferred_element_type=jnp.float32)
    gi = gi + bih_ref[...]
    gh = lax.dot_general(h, Whh, dn, preferred_element_type=jnp.float32)
    gh = gh + bhh_ref[...]
    r = jax.nn.sigmoid(gi[:, :DIM] + gh[:, :DIM])
    z = jax.nn.sigmoid(gi[:, DIM:2 * DIM] + gh[:, DIM:2 * DIM])
    n_ = jnp.tanh(gi[:, 2 * DIM:] + r * gh[:, 2 * DIM:])
    o_ref[...] = (1.0 - z) * n_ + z * h


def _gru(p, q, h, Wih, Whh, bih, bhh):
    n = h.shape[0]
    bs_x = pl.BlockSpec((BM, DIM), lambda i: (i, 0))
    return pl.pallas_call(
        _gru_body,
        grid=(n // BM,),
        in_specs=[
            bs_x, bs_x, bs_x,
            pl.BlockSpec((3 * DIM, 2 * DIM), lambda i: (0, 0)),
            pl.BlockSpec((3 * DIM, DIM), lambda i: (0, 0)),
            pl.BlockSpec((1, 3 * DIM), lambda i: (0, 0)),
            pl.BlockSpec((1, 3 * DIM), lambda i: (0, 0)),
        ],
        out_specs=bs_x,
        out_shape=jax.ShapeDtypeStruct((n, DIM), jnp.float32),
    )(p, q, h, Wih, Whh, bih.reshape(1, -1), bhh.reshape(1, -1))


NC = 2
NS = 16
K = 128
R = 8960
NR = 6
NPAD = NR * R
DUMP = R
ZR = 40


EPC = NS * K


def _aggr(msg, src, dst, offs):
    mesh = plsc.VectorSubcoreMesh(core_axis_name="c", subcore_axis_name="s")
    cp = pltpu.CompilerParams()
    if "needs_layout_passes" in pltpu.CompilerParams.__dataclass_fields__:
        cp = dataclasses.replace(cp, needs_layout_passes=False)

    cap = src.shape[0]
    nchmax = cap // (NS * K)

    @functools.partial(
        pl.kernel,
        out_type=jax.ShapeDtypeStruct((NPAD, DIM), jnp.float32),
        mesh=mesh,
        compiler_params=cp,
        scratch_types=[
            pltpu.VMEM((K,), jnp.int32),
            pltpu.VMEM((K,), jnp.int32),
            pltpu.VMEM((K,), jnp.int32),
            pltpu.VMEM((K,), jnp.int32),
            pltpu.VMEM((K,), jnp.int32),
            pltpu.VMEM((K,), jnp.int32),
            pltpu.VMEM((K,), jnp.int32),
            pltpu.VMEM((K,), jnp.int32),
            pltpu.VMEM((K, DIM), jnp.float32),
            pltpu.VMEM((K, DIM), jnp.float32),
            pltpu.VMEM((K, DIM), jnp.float32),
            pltpu.VMEM((ZR, DIM), jnp.float32),
            pltpu.VMEM((16,), jnp.int32),
            pltpu.VMEM_SHARED((R + 8, DIM), jnp.float32),
            pltpu.SemaphoreType.DMA,
            pltpu.SemaphoreType.DMA,
            pltpu.SemaphoreType.DMA,
            pltpu.SemaphoreType.DMA,
            pltpu.SemaphoreType.DMA,
            pltpu.SemaphoreType.DMA,
            pltpu.SemaphoreType.DMA,
            pltpu.SemaphoreType.DMA,
        ],
    )
    def k(msg_hbm, src_hbm, dst_hbm, offs_hbm, out_hbm,
          srcv0, srcv1, srcv2, dstv0, dstv1, dstv2, dsts0, dsts1,
          rows0, rows1, rows2, zbuf, offs_s, acc,
          sem0, sem1, sem2, isem0, isem1, isem2, ssem0, ssem1):
        cid = lax.axis_index("c")
        sid = lax.axis_index("s")
        srcv = (srcv0, srcv1, srcv2)
        dstv = (dstv0, dstv1, dstv2)
        dsts = (dsts0, dsts1)
        rows = (rows0, rows1, rows2)
        sem = (sem0, sem1, sem2)
        isem = (isem0, isem1, isem2)
        ssem = (ssem0, ssem1)
        pltpu.sync_copy(offs_hbm, offs_s)
        zero16 = jnp.zeros((16,), jnp.float32)

        @pl.loop(0, ZR)
        def _(r0):
            @pl.loop(0, DIM, step=16)
            def _(c0):
                zbuf[r0, pl.ds(c0, 16)] = zero16

        @pl.loop(0, NR // NC)
        def _(rloc):
            rng = cid * (NR // NC) + rloc
            base = rng * R

            @pl.loop(0, R // NS // ZR)
            def _(j):
                pltpu.sync_copy(zbuf, acc.at[pl.ds(sid * (R // NS) + j * ZR, ZR)])

            plsc.subcore_barrier()

            offv = offs_s[pl.ds(0, 16)]
            lane = lax.broadcasted_iota(jnp.int32, (16,), 0)
            e_start = jnp.sum(jnp.where(lane == rng, offv, 0), axis=0)
            e_next = jnp.sum(jnp.where(lane == rng + 1, offv, 0), axis=0)
            ect = (e_next - e_start) // NS
            t0 = e_start + sid * ect
            nch = ect // K

            def issue_idx(ch, p):
                e0 = pl.multiple_of(t0 + ch * K, K)
                pltpu.async_copy(src_hbm.at[pl.ds(e0, K)], srcv[p], isem[p])
                pltpu.async_copy(dst_hbm.at[pl.ds(e0, K)], dstv[p], isem[p])

            def wait_idx(ch, p):
                e0 = pl.multiple_of(t0 + ch * K, K)
                pltpu.make_async_copy(
                    src_hbm.at[pl.ds(e0, K)], srcv[p], isem[p]).wait()
                pltpu.make_async_copy(
                    dst_hbm.at[pl.ds(e0, K)], dstv[p], isem[p]).wait()

            def wait_scatter(q):
                pltpu.make_async_copy(
                    rows[0], acc.at[dsts[q]], ssem[q]).wait()

            @pl.when(nch > 0)
            def _():
                issue_idx(0, 0)

                @pl.when(nch > 1)
                def _():
                    issue_idx(1, 1)

                @pl.when(nch > 2)
                def _():
                    issue_idx(2, 2)

                wait_idx(0, 0)
                pltpu.async_copy(msg_hbm.at[srcv[0]], rows[0], sem[0])

            @pl.loop(0, nchmax + 5, step=6)
            def _(g):
                for u in range(6):
                    ch = g + u
                    p = u % 3
                    q = u % 2
                    p1 = (u + 1) % 3

                    @pl.when(ch < nch)
                    def _():
                        pltpu.make_async_copy(
                            msg_hbm.at[srcv[p]], rows[p], sem[p]).wait()

                        @pl.when(ch >= 2)
                        def _():
                            wait_scatter(q)

                        @pl.loop(0, K, step=16)
                        def _(i):
                            dsts[q][pl.ds(i, 16)] = dstv[p][pl.ds(i, 16)]
                        pltpu.async_copy(
                            rows[p], acc.at[dsts[q]], ssem[q], add=True)

                    @pl.when(ch + 1 < nch)
                    def _():
                        wait_idx(ch + 1, p1)
                        pltpu.async_copy(
                            msg_hbm.at[srcv[p1]], rows[p1], sem[p1])

                    @pl.when(ch + 3 < nch)
                    def _():
                        issue_idx(ch + 3, p)

            for qq in range(2):
                @pl.when(nch > qq)
                def _():
                    wait_scatter(qq)

            plsc.subcore_barrier()

            @pl.loop(0, R // NS // ZR)
            def _(j):
                r0 = sid * (R // NS) + j * ZR
                pltpu.sync_copy(acc.at[pl.ds(r0, ZR)], out_hbm.at[pl.ds(base + r0, ZR)])

    return k(msg, src, dst, offs)


def _partition(src, dst):
    e = src.shape[0]
    cap = -(-e // EPC) * EPC + NR * EPC
    b = dst // R
    counts = jnp.bincount(b, length=NR)
    cnt_pad = (-(-counts // EPC) * EPC).astype(jnp.int32)
    off = jnp.concatenate(
        [jnp.zeros((1,), jnp.int32), jnp.cumsum(cnt_pad).astype(jnp.int32)])
    rank = jnp.zeros((e,), jnp.int32)
    for kk in range(NR):
        m = b == kk
        rank = jnp.where(m, jnp.cumsum(m.astype(jnp.int32)) - 1, rank)
    pos = off[b] + rank
    packed = src * 16384 + (dst - b * R)
    packed_p = jnp.full((cap,), R, jnp.int32).at[pos].set(
        packed, unique_indices=True, mode="promise_in_bounds")
    src_p = packed_p >> 14
    dst_p = packed_p & 16383
    offs = jnp.concatenate([off, jnp.zeros((16 - NR - 1,), jnp.int32)])
    return src_p, dst_p, offs


def kernel(v_size, c_size, v_edge_index, c_edge_index, p_edge_index,
           n_edge_index, v_emb, c_emb, pv2c_W, pv2c_b, nv2c_W, nv2c_b,
           pc2v_W, pc2v_b, nc2v_W, nc2v_b, c_Wih, c_Whh, c_bih, c_bhh,
           v_Wih, v_Whh, v_bih, v_bhh):
    num_v = v_emb.shape[0]
    num_c = c_emb.shape[0]
    v_p = v_edge_index[p_edge_index]
    v_n = v_edge_index[n_edge_index]
    c_p = c_edge_index[p_edge_index]
    c_n = c_edge_index[n_edge_index]
    ed_pv2c = _partition(v_p, c_p)
    ed_nv2c = _partition(v_n, c_n)
    ed_pc2v = _partition(c_p, v_p)
    ed_nc2v = _partition(c_n, v_n)

    v_embs = [v_emb]
    c_embs = [c_emb]
    for _ in range(N_ITER):
        mv_p, mv_n = _mlp2(v_emb, pv2c_W, pv2c_b, nv2c_W, nv2c_b)
        mc_p, mc_n = _mlp2(c_emb, pc2v_W, pc2v_b, nc2v_W, nc2v_b)
        a_pv2c = _aggr(mv_p, *ed_pv2c)[:num_c]
        a_nv2c = _aggr(mv_n, *ed_nv2c)[:num_c]
        a_pc2v = _aggr(mc_p, *ed_pc2v)[:num_v]
        a_nc2v = _aggr(mc_n, *ed_nc2v)[:num_v]
        c_emb = _gru(a_pv2c, a_nv2c, c_emb, c_Wih, c_Whh, c_bih, c_bhh)
        v_emb = _gru(a_pc2v, a_nc2v, v_emb, v_Wih, v_Whh, v_bih, v_bhh)
        c_embs.append(c_emb)
        v_embs.append(v_emb)
    return jnp.stack(v_embs), jnp.stack(c_embs)


# speedup vs baseline: 1.0238x; 1.0238x over previous
import dataclasses
import functools

import jax
import jax.numpy as jnp
from jax import lax
from jax.experimental import pallas as pl
from jax.experimental.pallas import tpu as pltpu
from jax.experimental.pallas import tpu_sc as plsc

DIM = 128
N_LAYERS = 3
N_ITER = 4


BM = 1000


def _mlp2_body(x_ref, Wp_ref, bp_ref, Wn_ref, bn_ref, op_ref, on_ref):
    x = x_ref[...]

    def run(W_ref, b_ref, o_ref):
        h = x
        for i in range(N_LAYERS):
            h = lax.dot_general(h, W_ref[i], (((1,), (1,)), ((), ())),
                                preferred_element_type=jnp.float32)
            h = h + b_ref[i][None, :]
            if i < N_LAYERS - 1:
                h = jnp.maximum(h, 0.0)
        o_ref[...] = h

    run(Wp_ref, bp_ref, op_ref)
    run(Wn_ref, bn_ref, on_ref)


def _mlp2(x, Wp, bp, Wn, bn):
    n = x.shape[0]
    bs_x = pl.BlockSpec((BM, DIM), lambda i: (i, 0))
    bs_w = pl.BlockSpec((N_LAYERS, DIM, DIM), lambda i: (0, 0, 0))
    bs_b = pl.BlockSpec((N_LAYERS, DIM), lambda i: (0, 0))
    return pl.pallas_call(
        _mlp2_body,
        grid=(n // BM,),
        in_specs=[bs_x, bs_w, bs_b, bs_w, bs_b],
        out_specs=[bs_x, bs_x],
        out_shape=[jax.ShapeDtypeStruct((n, DIM), jnp.float32)] * 2,
    )(x, Wp, bp, Wn, bn)


def _gru_body(p_ref, n_ref, h_ref, Wih_ref, Whh_ref, bih_ref, bhh_ref, o_ref):
    p = p_ref[...]
    q = n_ref[...]
    h = h_ref[...]
    Wih = Wih_ref[...]
    Whh = Whh_ref[...]
    dn = (((1,), (1,)), ((), ()))
    gi = lax.dot_general(p, Wih[:, :DIM], dn, preferred_element_type=jnp.float32)
    gi = gi + lax.dot_general(q, Wih[:, DIM:], dn, preferred_element_type=jnp.float32)
    gi = gi + bih_ref[...]
    gh = lax.dot_general(h, Whh, dn, preferred_element_type=jnp.float32)
    gh = gh + bhh_ref[...]
    r = jax.nn.sigmoid(gi[:, :DIM] + gh[:, :DIM])
    z = jax.nn.sigmoid(gi[:, DIM:2 * DIM] + gh[:, DIM:2 * DIM])
    n_ = jnp.tanh(gi[:, 2 * DIM:] + r * gh[:, 2 * DIM:])
    o_ref[...] = (1.0 - z) * n_ + z * h


def _gru(p, q, h, Wih, Whh, bih, bhh):
    n = h.shape[0]
    bs_x = pl.BlockSpec((BM, DIM), lambda i: (i, 0))
    return pl.pallas_call(
        _gru_body,
        grid=(n // BM,),
        in_specs=[
            bs_x, bs_x, bs_x,
            pl.BlockSpec((3 * DIM, 2 * DIM), lambda i: (0, 0)),
            pl.BlockSpec((3 * DIM, DIM), lambda i: (0, 0)),
            pl.BlockSpec((1, 3 * DIM), lambda i: (0, 0)),
            pl.BlockSpec((1, 3 * DIM), lambda i: (0, 0)),
        ],
        out_specs=bs_x,
        out_shape=jax.ShapeDtypeStruct((n, DIM), jnp.float32),
    )(p, q, h, Wih, Whh, bih.reshape(1, -1), bhh.reshape(1, -1))


NC = 2
NS = 16
K = 128
R = 8960
NR = 6
NPAD = NR * R
DUMP = R
ZR = 80


EPC = NS * K


def _aggr(msg, src, dst, offs):
    mesh = plsc.VectorSubcoreMesh(core_axis_name="c", subcore_axis_name="s")
    cp = pltpu.CompilerParams()
    if "needs_layout_passes" in pltpu.CompilerParams.__dataclass_fields__:
        cp = dataclasses.replace(cp, needs_layout_passes=False)

    cap = src.shape[0]
    nchmax = cap // (NS * K)

    @functools.partial(
        pl.kernel,
        out_type=jax.ShapeDtypeStruct((NPAD, DIM), jnp.float32),
        mesh=mesh,
        compiler_params=cp,
        scratch_types=[
            pltpu.VMEM((K,), jnp.int32),
            pltpu.VMEM((K,), jnp.int32),
            pltpu.VMEM((K,), jnp.int32),
            pltpu.VMEM((K,), jnp.int32),
            pltpu.VMEM((K, DIM), jnp.float32),
            pltpu.VMEM((K, DIM), jnp.float32),
            pltpu.VMEM((ZR, DIM), jnp.float32),
            pltpu.VMEM((16,), jnp.int32),
            pltpu.VMEM_SHARED((R + 8, DIM), jnp.float32),
            pltpu.SemaphoreType.DMA,
            pltpu.SemaphoreType.DMA,
            pltpu.SemaphoreType.DMA,
            pltpu.SemaphoreType.DMA,
        ],
    )
    def k(msg_hbm, src_hbm, dst_hbm, offs_hbm, out_hbm,
          srcv0, srcv1, dstv0, dstv1, rows0, rows1, zbuf, offs_s, acc,
          sem0, sem1, isem0, isem1):
        cid = lax.axis_index("c")
        sid = lax.axis_index("s")
        srcv = (srcv0, srcv1)
        dstv = (dstv0, dstv1)
        rows = (rows0, rows1)
        sem = (sem0, sem1)
        isem = (isem0, isem1)
        pltpu.sync_copy(offs_hbm, offs_s)
        zero16 = jnp.zeros((16,), jnp.float32)

        @pl.loop(0, ZR)
        def _(r0):
            @pl.loop(0, DIM, step=16)
            def _(c0):
                zbuf[r0, pl.ds(c0, 16)] = zero16

        @pl.loop(0, NR // NC)
        def _(rloc):
            rng = cid * (NR // NC) + rloc
            base = rng * R

            @pl.loop(0, R // NS // ZR)
            def _(j):
                pltpu.sync_copy(zbuf, acc.at[pl.ds(sid * (R // NS) + j * ZR, ZR)])

            plsc.subcore_barrier()

            offv = offs_s[pl.ds(0, 16)]
            lane = lax.broadcasted_iota(jnp.int32, (16,), 0)
            e_start = jnp.sum(jnp.where(lane == rng, offv, 0), axis=0)
            e_next = jnp.sum(jnp.where(lane == rng + 1, offv, 0), axis=0)
            ect = (e_next - e_start) // NS
            t0 = e_start + sid * ect
            nch = ect // K

            def issue_idx(ch, b):
                e0 = pl.multiple_of(t0 + ch * K, K)
                pltpu.async_copy(src_hbm.at[pl.ds(e0, K)], srcv[b], isem[b])
                pltpu.async_copy(dst_hbm.at[pl.ds(e0, K)], dstv[b], isem[b])

            def wait_idx(ch, b):
                e0 = pl.multiple_of(t0 + ch * K, K)
                pltpu.make_async_copy(
                    src_hbm.at[pl.ds(e0, K)], srcv[b], isem[b]).wait()
                pltpu.make_async_copy(
                    dst_hbm.at[pl.ds(e0, K)], dstv[b], isem[b]).wait()

            @pl.when(nch > 0)
            def _():
                issue_idx(0, 0)

                @pl.when(nch > 1)
                def _():
                    issue_idx(1, 1)

                wait_idx(0, 0)
                pltpu.async_copy(msg_hbm.at[srcv[0]], rows[0], sem[0])

            @pl.loop(0, nchmax, step=2)
            def _(g):
                for b in range(2):
                    ch = g + b

                    @pl.when(ch + 1 < nch)
                    def _():
                        wait_idx(ch + 1, 1 - b)
                        pltpu.async_copy(
                            msg_hbm.at[srcv[1 - b]], rows[1 - b], sem[1 - b])

                    @pl.when(ch < nch)
                    def _():
                        pltpu.make_async_copy(
                            msg_hbm.at[srcv[b]], rows[b], sem[b]).wait()
                        pltpu.sync_copy(rows[b], acc.at[dstv[b]], add=True)

                    @pl.when(ch + 2 < nch)
                    def _():
                        issue_idx(ch + 2, b)

            plsc.subcore_barrier()

            @pl.loop(0, R // NS // ZR)
            def _(j):
                r0 = sid * (R // NS) + j * ZR
                pltpu.sync_copy(acc.at[pl.ds(r0, ZR)], out_hbm.at[pl.ds(base + r0, ZR)])

    return k(msg, src, dst, offs)


def _partition(src, dst):
    e = src.shape[0]
    cap = -(-e // EPC) * EPC + NR * EPC
    b = dst // R
    counts = jnp.bincount(b, length=NR)
    cnt_pad = (-(-counts // EPC) * EPC).astype(jnp.int32)
    off = jnp.concatenate(
        [jnp.zeros((1,), jnp.int32), jnp.cumsum(cnt_pad).astype(jnp.int32)])
    rank = jnp.zeros((e,), jnp.int32)
    for kk in range(NR):
        m = b == kk
        rank = jnp.where(m, jnp.cumsum(m.astype(jnp.int32)) - 1, rank)
    pos = off[b] + rank
    packed = src * 16384 + (dst - b * R)
    packed_p = jnp.full((cap,), R, jnp.int32).at[pos].set(
        packed, unique_indices=True, mode="promise_in_bounds")
    src_p = packed_p >> 14
    dst_p = packed_p & 16383
    offs = jnp.concatenate([off, jnp.zeros((16 - NR - 1,), jnp.int32)])
    return src_p, dst_p, offs


def kernel(v_size, c_size, v_edge_index, c_edge_index, p_edge_index,
           n_edge_index, v_emb, c_emb, pv2c_W, pv2c_b, nv2c_W, nv2c_b,
           pc2v_W, pc2v_b, nc2v_W, nc2v_b, c_Wih, c_Whh, c_bih, c_bhh,
           v_Wih, v_Whh, v_bih, v_bhh):
    num_v = v_emb.shape[0]
    num_c = c_emb.shape[0]
    v_p = v_edge_index[p_edge_index]
    v_n = v_edge_index[n_edge_index]
    c_p = c_edge_index[p_edge_index]
    c_n = c_edge_index[n_edge_index]
    ed_pv2c = _partition(v_p, c_p)
    ed_nv2c = _partition(v_n, c_n)
    ed_pc2v = _partition(c_p, v_p)
    ed_nc2v = _partition(c_n, v_n)

    v_embs = [v_emb]
    c_embs = [c_emb]
    for _ in range(N_ITER):
        mv_p, mv_n = _mlp2(v_emb, pv2c_W, pv2c_b, nv2c_W, nv2c_b)
        mc_p, mc_n = _mlp2(c_emb, pc2v_W, pc2v_b, nc2v_W, nc2v_b)
        a_pv2c = _aggr(mv_p, *ed_pv2c)[:num_c]
        a_nv2c = _aggr(mv_n, *ed_nv2c)[:num_c]
        a_pc2v = _aggr(mc_p, *ed_pc2v)[:num_v]
        a_nc2v = _aggr(mc_n, *ed_nc2v)[:num_v]
        c_emb = _gru(a_pv2c, a_nv2c, c_emb, c_Wih, c_Whh, c_bih, c_bhh)
        v_emb = _gru(a_pc2v, a_nc2v, v_emb, v_Wih, v_Whh, v_bih, v_bhh)
        c_embs.append(c_emb)
        v_embs.append(v_emb)
    return jnp.stack(v_embs), jnp.stack(c_embs)


# speedup vs baseline: 1.0387x; 1.0145x over previous
import dataclasses
import functools

import jax
import jax.numpy as jnp
from jax import lax
from jax.experimental import pallas as pl
from jax.experimental.pallas import tpu as pltpu
from jax.experimental.pallas import tpu_sc as plsc

DIM = 128
N_LAYERS = 3
N_ITER = 4


BM = 1000


def _mlp2_body(x_ref, Wp_ref, bp_ref, Wn_ref, bn_ref, op_ref, on_ref):
    x = x_ref[...]

    def run(W_ref, b_ref, o_ref):
        h = x
        for i in range(N_LAYERS):
            h = lax.dot_general(h, W_ref[i], (((1,), (1,)), ((), ())),
                                preferred_element_type=jnp.float32)
            h = h + b_ref[i][None, :]
            if i < N_LAYERS - 1:
                h = jnp.maximum(h, 0.0)
        o_ref[...] = h

    run(Wp_ref, bp_ref, op_ref)
    run(Wn_ref, bn_ref, on_ref)


def _mlp2(x, Wp, bp, Wn, bn):
    n = x.shape[0]
    bs_x = pl.BlockSpec((BM, DIM), lambda i: (i, 0))
    bs_w = pl.BlockSpec((N_LAYERS, DIM, DIM), lambda i: (0, 0, 0))
    bs_b = pl.BlockSpec((N_LAYERS, DIM), lambda i: (0, 0))
    return pl.pallas_call(
        _mlp2_body,
        grid=(n // BM,),
        in_specs=[bs_x, bs_w, bs_b, bs_w, bs_b],
        out_specs=[bs_x, bs_x],
        out_shape=[jax.ShapeDtypeStruct((n, DIM), jnp.float32)] * 2,
    )(x, Wp, bp, Wn, bn)


def _gru_body(p_ref, n_ref, h_ref, Wih_ref, Whh_ref, bih_ref, bhh_ref, o_ref):
    p = p_ref[...]
    q = n_ref[...]
    h = h_ref[...]
    Wih = Wih_ref[...]
    Whh = Whh_ref[...]
    dn = (((1,), (1,)), ((), ()))
    gi = lax.dot_general(p, Wih[:, :DIM], dn, preferred_element_type=jnp.float32)
    gi = gi + lax.dot_general(q, Wih[:, DIM:], dn, preferred_element_type=jnp.float32)
    gi = gi + bih_ref[...]
    gh = lax.dot_general(h, Whh, dn, preferred_element_type=jnp.float32)
    gh = gh + bhh_ref[...]
    r = jax.nn.sigmoid(gi[:, :DIM] + gh[:, :DIM])
    z = jax.nn.sigmoid(gi[:, DIM:2 * DIM] + gh[:, DIM:2 * DIM])
    n_ = jnp.tanh(gi[:, 2 * DIM:] + r * gh[:, 2 * DIM:])
    o_ref[...] = (1.0 - z) * n_ + z * h


def _gru(p, q, h, Wih, Whh, bih, bhh):
    n = h.shape[0]
    bs_x = pl.BlockSpec((BM, DIM), lambda i: (i, 0))
    return pl.pallas_call(
        _gru_body,
        grid=(n // BM,),
        in_specs=[
            bs_x, bs_x, bs_x,
            pl.BlockSpec((3 * DIM, 2 * DIM), lambda i: (0, 0)),
            pl.BlockSpec((3 * DIM, DIM), lambda i: (0, 0)),
            pl.BlockSpec((1, 3 * DIM), lambda i: (0, 0)),
            pl.BlockSpec((1, 3 * DIM), lambda i: (0, 0)),
        ],
        out_specs=bs_x,
        out_shape=jax.ShapeDtypeStruct((n, DIM), jnp.float32),
    )(p, q, h, Wih, Whh, bih.reshape(1, -1), bhh.reshape(1, -1))


NC = 2
NS = 16
K = 128
R = 8960
NR = 6
NPAD = NR * R
DUMP = R
ZR = 80


EPC = NS * K


def _aggr(msg, src, dst, offs):
    mesh = plsc.VectorSubcoreMesh(core_axis_name="c", subcore_axis_name="s")
    cp = pltpu.CompilerParams()
    if "needs_layout_passes" in pltpu.CompilerParams.__dataclass_fields__:
        cp = dataclasses.replace(cp, needs_layout_passes=False)

    cap = src.shape[0]
    nchmax = cap // (NS * K)

    @functools.partial(
        pl.kernel,
        out_type=jax.ShapeDtypeStruct((NPAD, DIM), jnp.float32),
        mesh=mesh,
        compiler_params=cp,
        scratch_types=[
            pltpu.VMEM((K,), jnp.int32),
            pltpu.VMEM((K,), jnp.int32),
            pltpu.VMEM((K,), jnp.int32),
            pltpu.VMEM((K,), jnp.int32),
            pltpu.VMEM((K, DIM), jnp.float32),
            pltpu.VMEM((K, DIM), jnp.float32),
            pltpu.VMEM((ZR, DIM), jnp.float32),
            pltpu.VMEM((16,), jnp.int32),
            pltpu.VMEM_SHARED((R + 8, DIM), jnp.float32),
            pltpu.SemaphoreType.DMA,
            pltpu.SemaphoreType.DMA,
            pltpu.SemaphoreType.DMA,
            pltpu.SemaphoreType.DMA,
        ],
    )
    def k(msg_hbm, src_hbm, dst_hbm, offs_hbm, out_hbm,
          srcv0, srcv1, dstv0, dstv1, rows0, rows1, zbuf, offs_s, acc,
          sem0, sem1, isem0, isem1):
        cid = lax.axis_index("c")
        sid = lax.axis_index("s")
        srcv = (srcv0, srcv1)
        dstv = (dstv0, dstv1)
        rows = (rows0, rows1)
        sem = (sem0, sem1)
        isem = (isem0, isem1)
        pltpu.sync_copy(offs_hbm, offs_s)
        zero16 = jnp.zeros((16,), jnp.float32)

        @pl.loop(0, ZR)
        def _(r0):
            @pl.loop(0, DIM, step=16)
            def _(c0):
                zbuf[r0, pl.ds(c0, 16)] = zero16

        @pl.loop(0, NR // NC)
        def _(rloc):
            rng = cid * (NR // NC) + rloc
            base = rng * R

            @pl.loop(0, R // NS // ZR)
            def _(j):
                pltpu.sync_copy(zbuf, acc.at[pl.ds(sid * (R // NS) + j * ZR, ZR)])

            plsc.subcore_barrier()

            offv = offs_s[pl.ds(0, 16)]
            lane = lax.broadcasted_iota(jnp.int32, (16,), 0)
            e_start = jnp.sum(jnp.where(lane == rng, offv, 0), axis=0)
            e_next = jnp.sum(jnp.where(lane == rng + 1, offv, 0), axis=0)
            ect = (e_next - e_start) // NS
            t0 = e_start + sid * ect
            nch = ect // K

            def issue_idx(ch, b):
                e0 = pl.multiple_of(t0 + ch * K, K)
                pltpu.async_copy(src_hbm.at[pl.ds(e0, K)], srcv[b], isem[b])
                pltpu.async_copy(dst_hbm.at[pl.ds(e0, K)], dstv[b], isem[b])

            def wait_idx(ch, b):
                e0 = pl.multiple_of(t0 + ch * K, K)
                pltpu.make_async_copy(
                    src_hbm.at[pl.ds(e0, K)], srcv[b], isem[b]).wait()
                pltpu.make_async_copy(
                    dst_hbm.at[pl.ds(e0, K)], dstv[b], isem[b]).wait()

            @pl.when(nch > 0)
            def _():
                issue_idx(0, 0)

                @pl.when(nch > 1)
                def _():
                    issue_idx(1, 1)

                wait_idx(0, 0)
                pltpu.async_copy(msg_hbm.at[srcv[0]], rows[0], sem[0])

            @pl.loop(0, nchmax, step=2)
            def _(g):
                for b in range(2):
                    ch = g + b

                    @pl.when(ch + 1 < nch)
                    def _():
                        wait_idx(ch + 1, 1 - b)
                        pltpu.async_copy(
                            msg_hbm.at[srcv[1 - b]], rows[1 - b], sem[1 - b])

                    @pl.when(ch < nch)
                    def _():
                        pltpu.make_async_copy(
                            msg_hbm.at[srcv[b]], rows[b], sem[b]).wait()
                        pltpu.sync_copy(rows[b], acc.at[dstv[b]], add=True)

                    @pl.when(ch + 2 < nch)
                    def _():
                        issue_idx(ch + 2, b)

            plsc.subcore_barrier()

            @pl.loop(0, R // NS // ZR)
            def _(j):
                r0 = sid * (R // NS) + j * ZR
                pltpu.sync_copy(acc.at[pl.ds(r0, ZR)], out_hbm.at[pl.ds(base + r0, ZR)])

    return k(msg, src, dst, offs)


def _partition(src, dst):
    e = src.shape[0]
    cap = -(-e // EPC) * EPC + NR * EPC
    b = dst // R
    counts = jnp.bincount(b, length=NR)
    cnt_pad = (-(-counts // EPC) * EPC).astype(jnp.int32)
    off = jnp.concatenate(
        [jnp.zeros((1,), jnp.int32), jnp.cumsum(cnt_pad).astype(jnp.int32)])
    rank = jnp.zeros((e,), jnp.int32)
    for kk in range(NR):
        m = b == kk
        rank = jnp.where(m, jnp.cumsum(m.astype(jnp.int32)) - 1, rank)
    pos = off[b] + rank
    packed = src * 16384 + (dst - b * R)
    packed_p = jnp.full((cap,), R, jnp.int32).at[pos].set(
        packed, unique_indices=True, mode="promise_in_bounds")
    src_p = packed_p >> 14
    dst_p = packed_p & 16383
    offs = jnp.concatenate([off, jnp.zeros((16 - NR - 1,), jnp.int32)])
    return src_p, dst_p, offs


def kernel(v_size, c_size, v_edge_index, c_edge_index, p_edge_index,
           n_edge_index, v_emb, c_emb, pv2c_W, pv2c_b, nv2c_W, nv2c_b,
           pc2v_W, pc2v_b, nc2v_W, nc2v_b, c_Wih, c_Whh, c_bih, c_bhh,
           v_Wih, v_Whh, v_bih, v_bhh):
    num_v = v_emb.shape[0]
    num_c = c_emb.shape[0]
    v_p = v_edge_index[p_edge_index]
    v_n = v_edge_index[n_edge_index]
    c_p = c_edge_index[p_edge_index]
    c_n = c_edge_index[n_edge_index]
    ed_pv2c = _partition(v_p, c_p)
    ed_nv2c = _partition(v_n, c_n)
    ed_pc2v = _partition(c_p, v_p)
    ed_nc2v = _partition(c_n, v_n)

    v_embs = [v_emb]
    c_embs = [c_emb]
    for _ in range(N_ITER):
        mv_p, mv_n = _mlp2(v_emb, pv2c_W, pv2c_b, nv2c_W, nv2c_b)
        mc_p, mc_n = _mlp2(c_emb, pc2v_W, pc2v_b, nc2v_W, nc2v_b)
        a_pv2c = _aggr(mv_p, *ed_pv2c)
        a_nv2c = _aggr(mv_n, *ed_nv2c)
        a_pc2v = _aggr(mc_p, *ed_pc2v)
        a_nc2v = _aggr(mc_n, *ed_nc2v)
        c_emb = _gru(a_pv2c, a_nv2c, c_emb, c_Wih, c_Whh, c_bih, c_bhh)
        v_emb = _gru(a_pc2v, a_nc2v, v_emb, v_Wih, v_Whh, v_bih, v_bhh)
        c_embs.append(c_emb)
        v_embs.append(v_emb)
    return jnp.stack(v_embs), jnp.stack(c_embs)


# speedup vs baseline: 1.0566x; 1.0173x over previous
import dataclasses
import functools

import jax
import jax.numpy as jnp
from jax import lax
from jax.experimental import pallas as pl
from jax.experimental.pallas import tpu as pltpu
from jax.experimental.pallas import tpu_sc as plsc

DIM = 128
N_LAYERS = 3
N_ITER = 4


BM = 2000


def _mlp2_body(x_ref, Wp_ref, bp_ref, Wn_ref, bn_ref, op_ref, on_ref):
    x = x_ref[...]

    def run(W_ref, b_ref, o_ref):
        h = x
        for i in range(N_LAYERS):
            h = lax.dot_general(h, W_ref[i], (((1,), (1,)), ((), ())),
                                preferred_element_type=jnp.float32)
            h = h + b_ref[i][None, :]
            if i < N_LAYERS - 1:
                h = jnp.maximum(h, 0.0)
        o_ref[...] = h

    run(Wp_ref, bp_ref, op_ref)
    run(Wn_ref, bn_ref, on_ref)


def _mlp2(x, Wp, bp, Wn, bn):
    n = x.shape[0]
    bs_x = pl.BlockSpec((BM, DIM), lambda i: (i, 0))
    bs_w = pl.BlockSpec((N_LAYERS, DIM, DIM), lambda i: (0, 0, 0))
    bs_b = pl.BlockSpec((N_LAYERS, DIM), lambda i: (0, 0))
    return pl.pallas_call(
        _mlp2_body,
        grid=(n // BM,),
        in_specs=[bs_x, bs_w, bs_b, bs_w, bs_b],
        out_specs=[bs_x, bs_x],
        out_shape=[jax.ShapeDtypeStruct((n, DIM), jnp.float32)] * 2,
    )(x, Wp, bp, Wn, bn)


def _gru_body(p_ref, n_ref, h_ref, Wih_ref, Whh_ref, bih_ref, bhh_ref, o_ref):
    p = p_ref[...]
    q = n_ref[...]
    h = h_ref[...]
    Wih = Wih_ref[...]
    Whh = Whh_ref[...]
    dn = (((1,), (1,)), ((), ()))
    gi = lax.dot_general(p, Wih[:, :DIM], dn, preferred_element_type=jnp.float32)
    gi = gi + lax.dot_general(q, Wih[:, DIM:], dn, preferred_element_type=jnp.float32)
    gi = gi + bih_ref[...]
    gh = lax.dot_general(h, Whh, dn, preferred_element_type=jnp.float32)
    gh = gh + bhh_ref[...]
    r = jax.nn.sigmoid(gi[:, :DIM] + gh[:, :DIM])
    z = jax.nn.sigmoid(gi[:, DIM:2 * DIM] + gh[:, DIM:2 * DIM])
    n_ = jnp.tanh(gi[:, 2 * DIM:] + r * gh[:, 2 * DIM:])
    o_ref[...] = (1.0 - z) * n_ + z * h


def _gru(p, q, h, Wih, Whh, bih, bhh):
    n = h.shape[0]
    bs_x = pl.BlockSpec((BM, DIM), lambda i: (i, 0))
    return pl.pallas_call(
        _gru_body,
        grid=(n // BM,),
        in_specs=[
            bs_x, bs_x, bs_x,
            pl.BlockSpec((3 * DIM, 2 * DIM), lambda i: (0, 0)),
            pl.BlockSpec((3 * DIM, DIM), lambda i: (0, 0)),
            pl.BlockSpec((1, 3 * DIM), lambda i: (0, 0)),
            pl.BlockSpec((1, 3 * DIM), lambda i: (0, 0)),
        ],
        out_specs=bs_x,
        out_shape=jax.ShapeDtypeStruct((n, DIM), jnp.float32),
    )(p, q, h, Wih, Whh, bih.reshape(1, -1), bhh.reshape(1, -1))


NC = 2
NS = 16
K = 128
R = 8960
NR = 6
NPAD = NR * R
DUMP = R
ZR = 80


EPC = NS * K


def _aggr(msg, src, dst, offs):
    mesh = plsc.VectorSubcoreMesh(core_axis_name="c", subcore_axis_name="s")
    cp = pltpu.CompilerParams()
    if "needs_layout_passes" in pltpu.CompilerParams.__dataclass_fields__:
        cp = dataclasses.replace(cp, needs_layout_passes=False)

    cap = src.shape[0]
    nchmax = cap // (NS * K)

    @functools.partial(
        pl.kernel,
        out_type=jax.ShapeDtypeStruct((NPAD, DIM), jnp.float32),
        mesh=mesh,
        compiler_params=cp,
        scratch_types=[
            pltpu.VMEM((K,), jnp.int32),
            pltpu.VMEM((K,), jnp.int32),
            pltpu.VMEM((K,), jnp.int32),
            pltpu.VMEM((K,), jnp.int32),
            pltpu.VMEM((K, DIM), jnp.float32),
            pltpu.VMEM((K, DIM), jnp.float32),
            pltpu.VMEM((ZR, DIM), jnp.float32),
            pltpu.VMEM((16,), jnp.int32),
            pltpu.VMEM_SHARED((R + 8, DIM), jnp.float32),
            pltpu.SemaphoreType.DMA,
            pltpu.SemaphoreType.DMA,
            pltpu.SemaphoreType.DMA,
            pltpu.SemaphoreType.DMA,
        ],
    )
    def k(msg_hbm, src_hbm, dst_hbm, offs_hbm, out_hbm,
          srcv0, srcv1, dstv0, dstv1, rows0, rows1, zbuf, offs_s, acc,
          sem0, sem1, isem0, isem1):
        cid = lax.axis_index("c")
        sid = lax.axis_index("s")
        srcv = (srcv0, srcv1)
        dstv = (dstv0, dstv1)
        rows = (rows0, rows1)
        sem = (sem0, sem1)
        isem = (isem0, isem1)
        pltpu.sync_copy(offs_hbm, offs_s)
        zero16 = jnp.zeros((16,), jnp.float32)

        @pl.loop(0, ZR)
        def _(r0):
            @pl.loop(0, DIM, step=16)
            def _(c0):
                zbuf[r0, pl.ds(c0, 16)] = zero16

        @pl.loop(0, NR // NC)
        def _(rloc):
            rng = cid * (NR // NC) + rloc
            base = rng * R

            @pl.loop(0, R // NS // ZR)
            def _(j):
                pltpu.sync_copy(zbuf, acc.at[pl.ds(sid * (R // NS) + j * ZR, ZR)])

            plsc.subcore_barrier()

            offv = offs_s[pl.ds(0, 16)]
            lane = lax.broadcasted_iota(jnp.int32, (16,), 0)
            e_start = jnp.sum(jnp.where(lane == rng, offv, 0), axis=0)
            e_next = jnp.sum(jnp.where(lane == rng + 1, offv, 0), axis=0)
            ect = (e_next - e_start) // NS
            t0 = e_start + sid * ect
            nch = ect // K

            def issue_idx(ch, b):
                e0 = pl.multiple_of(t0 + ch * K, K)
                pltpu.async_copy(src_hbm.at[pl.ds(e0, K)], srcv[b], isem[b])
                pltpu.async_copy(dst_hbm.at[pl.ds(e0, K)], dstv[b], isem[b])

            def wait_idx(ch, b):
                e0 = pl.multiple_of(t0 + ch * K, K)
                pltpu.make_async_copy(
                    src_hbm.at[pl.ds(e0, K)], srcv[b], isem[b]).wait()
                pltpu.make_async_copy(
                    dst_hbm.at[pl.ds(e0, K)], dstv[b], isem[b]).wait()

            @pl.when(nch > 0)
            def _():
                issue_idx(0, 0)

                @pl.when(nch > 1)
                def _():
                    issue_idx(1, 1)

                wait_idx(0, 0)
                pltpu.async_copy(msg_hbm.at[srcv[0]], rows[0], sem[0])

            @pl.loop(0, nchmax, step=2)
            def _(g):
                for b in range(2):
                    ch = g + b

                    @pl.when(ch + 1 < nch)
                    def _():
                        wait_idx(ch + 1, 1 - b)
                        pltpu.async_copy(
                            msg_hbm.at[srcv[1 - b]], rows[1 - b], sem[1 - b])

                    @pl.when(ch < nch)
                    def _():
                        pltpu.make_async_copy(
                            msg_hbm.at[srcv[b]], rows[b], sem[b]).wait()
                        pltpu.sync_copy(rows[b], acc.at[dstv[b]], add=True)

                    @pl.when(ch + 2 < nch)
                    def _():
                        issue_idx(ch + 2, b)

            plsc.subcore_barrier()

            @pl.loop(0, R // NS // ZR)
            def _(j):
                r0 = sid * (R // NS) + j * ZR
                pltpu.sync_copy(acc.at[pl.ds(r0, ZR)], out_hbm.at[pl.ds(base + r0, ZR)])

    return k(msg, src, dst, offs)


def _partition(src, dst):
    e = src.shape[0]
    cap = -(-e // EPC) * EPC + NR * EPC
    b = dst // R
    counts = jnp.bincount(b, length=NR)
    cnt_pad = (-(-counts // EPC) * EPC).astype(jnp.int32)
    off = jnp.concatenate(
        [jnp.zeros((1,), jnp.int32), jnp.cumsum(cnt_pad).astype(jnp.int32)])
    rank = jnp.zeros((e,), jnp.int32)
    for kk in range(NR):
        m = b == kk
        rank = jnp.where(m, jnp.cumsum(m.astype(jnp.int32)) - 1, rank)
    pos = off[b] + rank
    packed = src * 16384 + (dst - b * R)
    packed_p = jnp.full((cap,), R, jnp.int32).at[pos].set(
        packed, unique_indices=True, mode="promise_in_bounds")
    src_p = packed_p >> 14
    dst_p = packed_p & 16383
    offs = jnp.concatenate([off, jnp.zeros((16 - NR - 1,), jnp.int32)])
    return src_p, dst_p, offs


def kernel(v_size, c_size, v_edge_index, c_edge_index, p_edge_index,
           n_edge_index, v_emb, c_emb, pv2c_W, pv2c_b, nv2c_W, nv2c_b,
           pc2v_W, pc2v_b, nc2v_W, nc2v_b, c_Wih, c_Whh, c_bih, c_bhh,
           v_Wih, v_Whh, v_bih, v_bhh):
    num_v = v_emb.shape[0]
    num_c = c_emb.shape[0]
    v_p = v_edge_index[p_edge_index]
    v_n = v_edge_index[n_edge_index]
    c_p = c_edge_index[p_edge_index]
    c_n = c_edge_index[n_edge_index]
    ed_pv2c = _partition(v_p, c_p)
    ed_nv2c = _partition(v_n, c_n)
    ed_pc2v = _partition(c_p, v_p)
    ed_nc2v = _partition(c_n, v_n)

    v_embs = [v_emb]
    c_embs = [c_emb]
    for _ in range(N_ITER):
        mv_p, mv_n = _mlp2(v_emb, pv2c_W, pv2c_b, nv2c_W, nv2c_b)
        mc_p, mc_n = _mlp2(c_emb, pc2v_W, pc2v_b, nc2v_W, nc2v_b)
        a_pv2c = _aggr(mv_p, *ed_pv2c)
        a_nv2c = _aggr(mv_n, *ed_nv2c)
        a_pc2v = _aggr(mc_p, *ed_pc2v)
        a_nc2v = _aggr(mc_n, *ed_nc2v)
        c_emb = _gru(a_pv2c, a_nv2c, c_emb, c_Wih, c_Whh, c_bih, c_bhh)
        v_emb = _gru(a_pc2v, a_nc2v, v_emb, v_Wih, v_Whh, v_bih, v_bhh)
        c_embs.append(c_emb)
        v_embs.append(v_emb)
    return jnp.stack(v_embs), jnp.stack(c_embs)
